# baseline (speedup 1.0000x reference)
import functools

import numpy as np
import jax
import jax.numpy as jnp
from jax.experimental import pallas as pl
from jax.experimental.pallas import tpu as pltpu

N = 100000
H = 128
NX = 512
BLK = 25000
NBLK = N // BLK


def _threefry2x32(k1, k2, c1, c2):
    def rotl(v, r):
        return ((v << np.uint32(r)) | (v >> np.uint32(32 - r))).astype(np.uint32)
    ks = [np.uint32(k1), np.uint32(k2),
          np.uint32(np.uint32(k1) ^ np.uint32(k2) ^ np.uint32(0x1BD11BDA))]
    x0 = (c1 + ks[0]).astype(np.uint32)
    x1 = (c2 + ks[1]).astype(np.uint32)
    rotations = [(13, 15, 26, 6), (17, 29, 16, 24)]
    for i in range(5):
        for r in rotations[i % 2]:
            x0 = (x0 + x1).astype(np.uint32)
            x1 = rotl(x1, r)
            x1 = (x1 ^ x0).astype(np.uint32)
        x0 = (x0 + ks[(i + 1) % 3]).astype(np.uint32)
        x1 = (x1 + ks[(i + 2) % 3] + np.uint32(i + 1)).astype(np.uint32)
    return x0, x1


def _gumbel_const(seed: int, n: int) -> np.ndarray:
    idx = np.arange(n, dtype=np.uint64)
    o0, o1 = _threefry2x32(np.uint32(seed >> 32), np.uint32(seed & 0xFFFFFFFF),
                           (idx >> np.uint64(32)).astype(np.uint32),
                           (idx & np.uint64(0xFFFFFFFF)).astype(np.uint32))
    bits = o0 ^ o1
    f = ((bits >> np.uint32(9)) | np.float32(1.0).view(np.uint32)).view(np.float32) \
        - np.float32(1.0)
    tiny = np.float32(np.finfo(np.float32).tiny)
    u = np.maximum(tiny, f + tiny)
    return (-np.log(-np.log(u))).astype(np.float32)


_G1 = _gumbel_const(1, N)
_G2 = _gumbel_const(2, NX)


def _ac_kernel(gea_ref, geb_ref, g1a_ref, g1b_ref, wc1_ref, bc1t_ref,
               wc2t_ref, bc2_ref,
               w1_ref, b1_ref, w2_ref, b2_ref, mask_ref, g2_ref,
               node_ref, value_ref, xfer_ref, xlp_ref, xent_ref,
               max_ref, row_ref):
    i = pl.program_id(0)

    @pl.when(i == 0)
    def _init():
        value_ref[0, 0] = 0.0
        max_ref[0, 0] = -jnp.inf
        node_ref[0, 0] = 0

    def _half(ge_ref, g1_ref, blk_idx):
        x = ge_ref[...]
        ht = jnp.maximum(
            jax.lax.dot_general(wc1_ref[...], x, (((0,), (1,)), ((), ())),
                                preferred_element_type=jnp.float32)
            + bc1t_ref[...], 0.0)
        v = jax.lax.dot_general(
            wc2t_ref[...], ht, (((1,), (0,)), ((), ())),
            preferred_element_type=jnp.float32) + bc2_ref[0, 0]

        value_ref[0, 0] += jnp.sum(v)

        score = v + g1_ref[0]
        lmax = jnp.max(score)

        @pl.when(lmax > max_ref[0, 0])
        def _update():
            max_ref[0, 0] = lmax
            cidx = jax.lax.broadcasted_iota(jnp.int32, (1, BLK), 1)
            larg = jnp.min(jnp.where(score == lmax, cidx, BLK))
            node_ref[0, 0] = blk_idx * BLK + larg
            row_ref[...] = ge_ref[pl.ds(larg, 1), :]

    _half(gea_ref, g1a_ref, 2 * i)
    _half(geb_ref, g1b_ref, 2 * i + 1)

    @pl.when(i == NBLK // 2 - 1)
    def _finish():
        row = row_ref[...]
        ha = jnp.maximum(
            jnp.dot(row, w1_ref[...], preferred_element_type=jnp.float32)
            + b1_ref[...], 0.0)
        logits = (jnp.dot(ha, w2_ref[...], preferred_element_type=jnp.float32)
                  + b2_ref[...])
        masked = jnp.where(mask_ref[...], logits, logits - 1e10)
        m = jnp.max(masked)
        shifted = masked - m
        lse = jnp.log(jnp.sum(jnp.exp(shifted)))
        logp = shifted - lse

        xs = masked + g2_ref[...]
        xmax = jnp.max(xs)
        cidx = jax.lax.broadcasted_iota(jnp.int32, (1, NX), 1)
        xarg = jnp.min(jnp.where(xs == xmax, cidx, NX))
        xfer_ref[0, 0] = xarg
        xlp_ref[0, 0] = jnp.sum(jnp.where(cidx == xarg, logp, 0.0))
        xent_ref[0, 0] = -jnp.sum(jnp.exp(logp) * logp)


@functools.partial(jax.jit, static_argnames=())
def kernel(graph_embed, mask, W1, b1, W2, b2, Wc1, bc1, Wc2, bc2):
    g1 = jnp.asarray(_G1.reshape(NBLK, 1, BLK))
    g2 = jnp.asarray(_G2.reshape(1, NX))

    grid = (NBLK // 2,)
    scal = pl.BlockSpec(memory_space=pltpu.SMEM)
    full = lambda shape: pl.BlockSpec(shape, lambda i: (0,) * len(shape))
    out = pl.pallas_call(
        _ac_kernel,
        grid=grid,
        in_specs=[
            pl.BlockSpec((BLK, H), lambda i: (2 * i, 0)),
            pl.BlockSpec((BLK, H), lambda i: (2 * i + 1, 0)),
            pl.BlockSpec((1, 1, BLK), lambda i: (2 * i, 0, 0)),
            pl.BlockSpec((1, 1, BLK), lambda i: (2 * i + 1, 0, 0)),
            full((H, H // 2)),
            full((H // 2, 1)),
            full((1, H // 2)),
            full((1, 1)),
            full((H, 2 * H)),
            full((1, 2 * H)),
            full((2 * H, NX)),
            full((1, NX)),
            full((1, NX)),
            full((1, NX)),
        ],
        out_specs=[scal, scal, scal, scal, scal],
        out_shape=[
            jax.ShapeDtypeStruct((1, 1), jnp.int32),
            jax.ShapeDtypeStruct((1, 1), jnp.float32),
            jax.ShapeDtypeStruct((1, 1), jnp.int32),
            jax.ShapeDtypeStruct((1, 1), jnp.float32),
            jax.ShapeDtypeStruct((1, 1), jnp.float32),
        ],
        scratch_shapes=[
            pltpu.SMEM((1, 1), jnp.float32),
            pltpu.VMEM((1, H), jnp.float32),
        ],
    )(
        graph_embed, graph_embed, g1, g1,
        Wc1, bc1.reshape(H // 2, 1), Wc2.reshape(1, H // 2), bc2.reshape(1, 1),
        W1, b1.reshape(1, 2 * H), W2, b2.reshape(1, NX),
        mask.reshape(1, NX), g2,
    )
    node, value, xfer, xlp, xent = out
    return (node[0, 0], xfer[0, 0], xlp[0, 0], xent[0, 0], value[0, 0])


# speedup vs baseline: 1.1557x; 1.1557x over previous
import functools

import numpy as np
import jax
import jax.numpy as jnp
from jax.experimental import pallas as pl
from jax.experimental.pallas import tpu as pltpu

N = 100000
H = 128
NX = 512
BLK = 20000
NBLK = N // BLK


def _threefry2x32(k1, k2, c1, c2):
    def rotl(v, r):
        return ((v << np.uint32(r)) | (v >> np.uint32(32 - r))).astype(np.uint32)
    ks = [np.uint32(k1), np.uint32(k2),
          np.uint32(np.uint32(k1) ^ np.uint32(k2) ^ np.uint32(0x1BD11BDA))]
    x0 = (c1 + ks[0]).astype(np.uint32)
    x1 = (c2 + ks[1]).astype(np.uint32)
    rotations = [(13, 15, 26, 6), (17, 29, 16, 24)]
    for i in range(5):
        for r in rotations[i % 2]:
            x0 = (x0 + x1).astype(np.uint32)
            x1 = rotl(x1, r)
            x1 = (x1 ^ x0).astype(np.uint32)
        x0 = (x0 + ks[(i + 1) % 3]).astype(np.uint32)
        x1 = (x1 + ks[(i + 2) % 3] + np.uint32(i + 1)).astype(np.uint32)
    return x0, x1


def _gumbel_const(seed: int, n: int) -> np.ndarray:
    idx = np.arange(n, dtype=np.uint64)
    o0, o1 = _threefry2x32(np.uint32(seed >> 32), np.uint32(seed & 0xFFFFFFFF),
                           (idx >> np.uint64(32)).astype(np.uint32),
                           (idx & np.uint64(0xFFFFFFFF)).astype(np.uint32))
    bits = o0 ^ o1
    f = ((bits >> np.uint32(9)) | np.float32(1.0).view(np.uint32)).view(np.float32) \
        - np.float32(1.0)
    tiny = np.float32(np.finfo(np.float32).tiny)
    u = np.maximum(tiny, f + tiny)
    return (-np.log(-np.log(u))).astype(np.float32)


_G1 = _gumbel_const(1, N)
_G2 = _gumbel_const(2, NX)


def _ac_kernel(ge_ref, g1_ref, wc1_ref, bc1t_ref, wc2t_ref, bc2_ref,
               w1_ref, b1_ref, w2_ref, b2_ref, mask_ref, g2_ref,
               node_ref, value_ref, xfer_ref, xlp_ref, xent_ref,
               max_ref, row_ref):
    i = pl.program_id(0)

    @pl.when(i == 0)
    def _init():
        value_ref[0, 0] = 0.0
        max_ref[0, 0] = -jnp.inf
        node_ref[0, 0] = 0

    x = ge_ref[...]
    ht = jnp.maximum(
        jax.lax.dot_general(wc1_ref[...], x, (((0,), (1,)), ((), ())),
                            preferred_element_type=jnp.float32)
        + bc1t_ref[...], 0.0)
    v = jax.lax.dot_general(
        wc2t_ref[...], ht, (((1,), (0,)), ((), ())),
        preferred_element_type=jnp.float32) + bc2_ref[0, 0]

    value_ref[0, 0] += jnp.sum(v)

    score = v + g1_ref[0]
    lmax = jnp.max(score)

    @pl.when(lmax > max_ref[0, 0])
    def _update():
        max_ref[0, 0] = lmax
        cidx = jax.lax.broadcasted_iota(jnp.int32, (1, BLK), 1)
        larg = jnp.min(jnp.where(score == lmax, cidx, BLK))
        node_ref[0, 0] = i * BLK + larg
        row_ref[...] = ge_ref[pl.ds(larg, 1), :]

    @pl.when(i == NBLK - 1)
    def _finish():
        row = row_ref[...]
        ha = jnp.maximum(
            jnp.dot(row, w1_ref[...], preferred_element_type=jnp.float32)
            + b1_ref[...], 0.0)
        logits = (jnp.dot(ha, w2_ref[...], preferred_element_type=jnp.float32)
                  + b2_ref[...])
        masked = jnp.where(mask_ref[...], logits, logits - 1e10)
        m = jnp.max(masked)
        shifted = masked - m
        lse = jnp.log(jnp.sum(jnp.exp(shifted)))
        logp = shifted - lse

        xs = masked + g2_ref[...]
        xmax = jnp.max(xs)
        cidx = jax.lax.broadcasted_iota(jnp.int32, (1, NX), 1)
        xarg = jnp.min(jnp.where(xs == xmax, cidx, NX))
        xfer_ref[0, 0] = xarg
        xlp_ref[0, 0] = jnp.sum(jnp.where(cidx == xarg, logp, 0.0))
        xent_ref[0, 0] = -jnp.sum(jnp.exp(logp) * logp)


@functools.partial(jax.jit, static_argnames=())
def kernel(graph_embed, mask, W1, b1, W2, b2, Wc1, bc1, Wc2, bc2):
    g1 = jnp.asarray(_G1.reshape(NBLK, 1, BLK))
    g2 = jnp.asarray(_G2.reshape(1, NX))

    grid = (NBLK,)
    scal = pl.BlockSpec(memory_space=pltpu.SMEM)
    full = lambda shape: pl.BlockSpec(shape, lambda i: (0,) * len(shape))
    out = pl.pallas_call(
        _ac_kernel,
        grid=grid,
        in_specs=[
            pl.BlockSpec((BLK, H), lambda i: (i, 0)),
            pl.BlockSpec((1, 1, BLK), lambda i: (i, 0, 0)),
            full((H, H // 2)),
            full((H // 2, 1)),
            full((1, H // 2)),
            full((1, 1)),
            full((H, 2 * H)),
            full((1, 2 * H)),
            full((2 * H, NX)),
            full((1, NX)),
            full((1, NX)),
            full((1, NX)),
        ],
        out_specs=[scal, scal, scal, scal, scal],
        out_shape=[
            jax.ShapeDtypeStruct((1, 1), jnp.int32),
            jax.ShapeDtypeStruct((1, 1), jnp.float32),
            jax.ShapeDtypeStruct((1, 1), jnp.int32),
            jax.ShapeDtypeStruct((1, 1), jnp.float32),
            jax.ShapeDtypeStruct((1, 1), jnp.float32),
        ],
        scratch_shapes=[
            pltpu.SMEM((1, 1), jnp.float32),
            pltpu.VMEM((1, H), jnp.float32),
        ],
    )(
        graph_embed, g1,
        Wc1, bc1.reshape(H // 2, 1), Wc2.reshape(1, H // 2), bc2.reshape(1, 1),
        W1, b1.reshape(1, 2 * H), W2, b2.reshape(1, NX),
        mask.reshape(1, NX), g2,
    )
    node, value, xfer, xlp, xent = out
    return (node[0, 0], xfer[0, 0], xlp[0, 0], xent[0, 0], value[0, 0])


# speedup vs baseline: 1.1590x; 1.0028x over previous
import functools

import numpy as np
import jax
import jax.numpy as jnp
from jax.experimental import pallas as pl
from jax.experimental.pallas import tpu as pltpu

N = 100000
H = 128
NX = 512
BLK = 25000
NBLK = N // BLK


def _threefry2x32(k1, k2, c1, c2):
    def rotl(v, r):
        return ((v << np.uint32(r)) | (v >> np.uint32(32 - r))).astype(np.uint32)
    ks = [np.uint32(k1), np.uint32(k2),
          np.uint32(np.uint32(k1) ^ np.uint32(k2) ^ np.uint32(0x1BD11BDA))]
    x0 = (c1 + ks[0]).astype(np.uint32)
    x1 = (c2 + ks[1]).astype(np.uint32)
    rotations = [(13, 15, 26, 6), (17, 29, 16, 24)]
    for i in range(5):
        for r in rotations[i % 2]:
            x0 = (x0 + x1).astype(np.uint32)
            x1 = rotl(x1, r)
            x1 = (x1 ^ x0).astype(np.uint32)
        x0 = (x0 + ks[(i + 1) % 3]).astype(np.uint32)
        x1 = (x1 + ks[(i + 2) % 3] + np.uint32(i + 1)).astype(np.uint32)
    return x0, x1


def _gumbel_const(seed: int, n: int) -> np.ndarray:
    idx = np.arange(n, dtype=np.uint64)
    o0, o1 = _threefry2x32(np.uint32(seed >> 32), np.uint32(seed & 0xFFFFFFFF),
                           (idx >> np.uint64(32)).astype(np.uint32),
                           (idx & np.uint64(0xFFFFFFFF)).astype(np.uint32))
    bits = o0 ^ o1
    f = ((bits >> np.uint32(9)) | np.float32(1.0).view(np.uint32)).view(np.float32) \
        - np.float32(1.0)
    tiny = np.float32(np.finfo(np.float32).tiny)
    u = np.maximum(tiny, f + tiny)
    return (-np.log(-np.log(u))).astype(np.float32)


_G1 = _gumbel_const(1, N)
_G2 = _gumbel_const(2, NX)


def _ac_kernel(ge_ref, g1_ref, wc1_ref, bc1t_ref, wc2t_ref, bc2_ref,
               w1_ref, b1_ref, w2_ref, b2_ref, mask_ref, g2_ref,
               node_ref, value_ref, xfer_ref, xlp_ref, xent_ref,
               max_ref, row_ref):
    i = pl.program_id(0)

    @pl.when(i == 0)
    def _init():
        value_ref[0, 0] = 0.0
        max_ref[0, 0] = -jnp.inf
        node_ref[0, 0] = 0

    x = ge_ref[...]
    ht = jnp.maximum(
        jax.lax.dot_general(wc1_ref[...], x, (((0,), (1,)), ((), ())),
                            preferred_element_type=jnp.float32)
        + bc1t_ref[...], 0.0)
    v = jax.lax.dot_general(
        wc2t_ref[...], ht, (((1,), (0,)), ((), ())),
        preferred_element_type=jnp.float32) + bc2_ref[0, 0]

    value_ref[0, 0] += jnp.sum(v)

    score = v + g1_ref[0]
    lmax = jnp.max(score)

    @pl.when(lmax > max_ref[0, 0])
    def _update():
        max_ref[0, 0] = lmax
        cidx = jax.lax.broadcasted_iota(jnp.int32, (1, BLK), 1)
        larg = jnp.min(jnp.where(score == lmax, cidx, BLK))
        node_ref[0, 0] = i * BLK + larg
        row_ref[...] = ge_ref[pl.ds(larg, 1), :]

    @pl.when(i == NBLK - 1)
    def _finish():
        row = row_ref[...]
        ha = jnp.maximum(
            jnp.dot(row, w1_ref[...], preferred_element_type=jnp.float32)
            + b1_ref[...], 0.0)
        logits = (jnp.dot(ha, w2_ref[...], preferred_element_type=jnp.float32)
                  + b2_ref[...])
        masked = jnp.where(mask_ref[...], logits, logits - 1e10)
        m = jnp.max(masked)
        shifted = masked - m
        lse = jnp.log(jnp.sum(jnp.exp(shifted)))
        logp = shifted - lse

        xs = masked + g2_ref[...]
        xmax = jnp.max(xs)
        cidx = jax.lax.broadcasted_iota(jnp.int32, (1, NX), 1)
        xarg = jnp.min(jnp.where(xs == xmax, cidx, NX))
        xfer_ref[0, 0] = xarg
        xlp_ref[0, 0] = jnp.sum(jnp.where(cidx == xarg, logp, 0.0))
        xent_ref[0, 0] = -jnp.sum(jnp.exp(logp) * logp)


@functools.partial(jax.jit, static_argnames=())
def kernel(graph_embed, mask, W1, b1, W2, b2, Wc1, bc1, Wc2, bc2):
    g1 = jnp.asarray(_G1.reshape(NBLK, 1, BLK))
    g2 = jnp.asarray(_G2.reshape(1, NX))

    grid = (NBLK,)
    scal = pl.BlockSpec(memory_space=pltpu.SMEM)
    full = lambda shape: pl.BlockSpec(shape, lambda i: (0,) * len(shape))
    out = pl.pallas_call(
        _ac_kernel,
        grid=grid,
        in_specs=[
            pl.BlockSpec((BLK, H), lambda i: (i, 0)),
            pl.BlockSpec((1, 1, BLK), lambda i: (i, 0, 0)),
            full((H, H // 2)),
            full((H // 2, 1)),
            full((1, H // 2)),
            full((1, 1)),
            full((H, 2 * H)),
            full((1, 2 * H)),
            full((2 * H, NX)),
            full((1, NX)),
            full((1, NX)),
            full((1, NX)),
        ],
        out_specs=[scal, scal, scal, scal, scal],
        out_shape=[
            jax.ShapeDtypeStruct((1, 1), jnp.int32),
            jax.ShapeDtypeStruct((1, 1), jnp.float32),
            jax.ShapeDtypeStruct((1, 1), jnp.int32),
            jax.ShapeDtypeStruct((1, 1), jnp.float32),
            jax.ShapeDtypeStruct((1, 1), jnp.float32),
        ],
        scratch_shapes=[
            pltpu.SMEM((1, 1), jnp.float32),
            pltpu.VMEM((1, H), jnp.float32),
        ],
    )(
        graph_embed, g1,
        Wc1, bc1.reshape(H // 2, 1), Wc2.reshape(1, H // 2), bc2.reshape(1, 1),
        W1, b1.reshape(1, 2 * H), W2, b2.reshape(1, NX),
        mask.reshape(1, NX), g2,
    )
    node, value, xfer, xlp, xent = out
    return (node[0, 0], xfer[0, 0], xlp[0, 0], xent[0, 0], value[0, 0])


# speedup vs baseline: 1.3060x; 1.1269x over previous
import functools

import numpy as np
import jax
import jax.numpy as jnp
from jax.experimental import pallas as pl
from jax.experimental.pallas import tpu as pltpu

N = 100000
H = 128
NX = 512
BLK = 25000
NBLK = N // BLK


def _threefry2x32(k1, k2, c1, c2):
    def rotl(v, r):
        return ((v << np.uint32(r)) | (v >> np.uint32(32 - r))).astype(np.uint32)
    ks = [np.uint32(k1), np.uint32(k2),
          np.uint32(np.uint32(k1) ^ np.uint32(k2) ^ np.uint32(0x1BD11BDA))]
    x0 = (c1 + ks[0]).astype(np.uint32)
    x1 = (c2 + ks[1]).astype(np.uint32)
    rotations = [(13, 15, 26, 6), (17, 29, 16, 24)]
    for i in range(5):
        for r in rotations[i % 2]:
            x0 = (x0 + x1).astype(np.uint32)
            x1 = rotl(x1, r)
            x1 = (x1 ^ x0).astype(np.uint32)
        x0 = (x0 + ks[(i + 1) % 3]).astype(np.uint32)
        x1 = (x1 + ks[(i + 2) % 3] + np.uint32(i + 1)).astype(np.uint32)
    return x0, x1


def _gumbel_const(seed: int, n: int) -> np.ndarray:
    idx = np.arange(n, dtype=np.uint64)
    o0, o1 = _threefry2x32(np.uint32(seed >> 32), np.uint32(seed & 0xFFFFFFFF),
                           (idx >> np.uint64(32)).astype(np.uint32),
                           (idx & np.uint64(0xFFFFFFFF)).astype(np.uint32))
    bits = o0 ^ o1
    f = ((bits >> np.uint32(9)) | np.float32(1.0).view(np.uint32)).view(np.float32) \
        - np.float32(1.0)
    tiny = np.float32(np.finfo(np.float32).tiny)
    u = np.maximum(tiny, f + tiny)
    return (-np.log(-np.log(u))).astype(np.float32)


_G1 = _gumbel_const(1, N)
_G2 = _gumbel_const(2, NX)


def _ac_kernel(ge_ref, g1_ref, wc1_ref, bc1r_ref, wc2t_ref, bc2_ref,
               w1_ref, b1_ref, w2_ref, b2_ref, mask_ref, g2_ref,
               node_ref, value_ref, xfer_ref, xlp_ref, xent_ref,
               max_ref, row_ref):
    i = pl.program_id(0)

    @pl.when(i == 0)
    def _init():
        value_ref[0, 0] = 0.0
        max_ref[0, 0] = -jnp.inf
        node_ref[0, 0] = 0

    x = ge_ref[...]
    bc1t = jax.lax.transpose(bc1r_ref[...], (1, 0))
    ht = jnp.maximum(
        jax.lax.dot_general(wc1_ref[...], x, (((1,), (1,)), ((), ())),
                            preferred_element_type=jnp.float32)
        + bc1t, 0.0)
    v = jax.lax.dot_general(
        wc2t_ref[...], ht, (((1,), (0,)), ((), ())),
        preferred_element_type=jnp.float32) + bc2_ref[0]

    value_ref[0, 0] += jnp.sum(v)

    score = v + g1_ref[0]
    lmax = jnp.max(score)

    @pl.when(lmax > max_ref[0, 0])
    def _update():
        max_ref[0, 0] = lmax
        cidx = jax.lax.broadcasted_iota(jnp.int32, (1, BLK), 1)
        larg = jnp.min(jnp.where(score == lmax, cidx, BLK))
        node_ref[0, 0] = i * BLK + larg
        row_ref[...] = ge_ref[pl.ds(larg, 1), :]

    @pl.when(i == NBLK - 1)
    def _finish():
        row = row_ref[...]
        ha = jnp.maximum(
            jnp.dot(row, w1_ref[...], preferred_element_type=jnp.float32)
            + b1_ref[...], 0.0)
        logits = (jnp.dot(ha, w2_ref[...], preferred_element_type=jnp.float32)
                  + b2_ref[...])
        masked = jnp.where(mask_ref[...], logits, logits - 1e10)
        m = jnp.max(masked)
        shifted = masked - m
        lse = jnp.log(jnp.sum(jnp.exp(shifted)))
        logp = shifted - lse

        xs = masked + g2_ref[...]
        xmax = jnp.max(xs)
        cidx = jax.lax.broadcasted_iota(jnp.int32, (1, NX), 1)
        xarg = jnp.min(jnp.where(xs == xmax, cidx, NX))
        xfer_ref[0, 0] = xarg
        xlp_ref[0, 0] = jnp.sum(jnp.where(cidx == xarg, logp, 0.0))
        xent_ref[0, 0] = -jnp.sum(jnp.exp(logp) * logp)


@functools.partial(jax.jit, static_argnames=())
def kernel(graph_embed, mask, W1, b1, W2, b2, Wc1, bc1, Wc2, bc2):
    g1 = jnp.asarray(_G1.reshape(NBLK, 1, BLK))
    g2 = jnp.asarray(_G2.reshape(1, NX))

    grid = (NBLK,)
    scal = pl.BlockSpec(memory_space=pltpu.SMEM)
    full = lambda shape: pl.BlockSpec(shape, lambda i: (0,) * len(shape))
    out = pl.pallas_call(
        _ac_kernel,
        grid=grid,
        in_specs=[
            pl.BlockSpec((BLK, H), lambda i: (i, 0)),
            pl.BlockSpec((1, 1, BLK), lambda i: (i, 0, 0)),
            full((H // 2, H)),
            full((1, H // 2)),
            full((1, H // 2)),
            pl.BlockSpec(memory_space=pltpu.SMEM),
            full((H, 2 * H)),
            full((1, 2 * H)),
            full((2 * H, NX)),
            full((1, NX)),
            full((1, NX)),
            full((1, NX)),
        ],
        out_specs=[scal, scal, scal, scal, scal],
        out_shape=[
            jax.ShapeDtypeStruct((1, 1), jnp.int32),
            jax.ShapeDtypeStruct((1, 1), jnp.float32),
            jax.ShapeDtypeStruct((1, 1), jnp.int32),
            jax.ShapeDtypeStruct((1, 1), jnp.float32),
            jax.ShapeDtypeStruct((1, 1), jnp.float32),
        ],
        scratch_shapes=[
            pltpu.SMEM((1, 1), jnp.float32),
            pltpu.VMEM((1, H), jnp.float32),
        ],
    )(
        graph_embed, g1,
        Wc1.T, bc1.reshape(1, H // 2), Wc2.reshape(1, H // 2), bc2,
        W1, b1.reshape(1, 2 * H), W2, b2.reshape(1, NX),
        mask.reshape(1, NX), g2,
    )
    node, value, xfer, xlp, xent = out
    return (node[0, 0], xfer[0, 0], xlp[0, 0], xent[0, 0], value[0, 0])
